# Initial kernel scaffold; baseline (speedup 1.0000x reference)
import jax
import jax.numpy as jnp
from jax.experimental import pallas as pl


def kernel(x, G_W1, G_b1, G_W2, G_b2, W1, b1, W2, b2):
    raise NotImplementedError("write your pallas kernel here")


# speedup vs baseline: 11.3743x; 11.3743x over previous
import functools

import jax
import jax.numpy as jnp
from jax.experimental import pallas as pl

_KP1 = 21
_NEG = -1e30


def _dot(a, b, trans_b=False):
    dn = (((1,), (1 if trans_b else 0,)), ((), ()))
    return jax.lax.dot_general(a, b, dn, preferred_element_type=jnp.float32)


def _embed_kernel(x_ref, gw1_ref, gb1_ref, gw2_ref, gb2_ref, w1_ref, b1_ref,
                  e_ref, xw1_ref):
    x = x_ref[...]
    z = jnp.maximum(_dot(x, gw1_ref[...]) + gb1_ref[...], 0.0)
    e = _dot(z, gw2_ref[...]) + gb2_ref[...]
    nrm = jnp.sqrt(jnp.sum(e * e, axis=1, keepdims=True))
    e_ref[...] = e / (nrm + 1e-12)
    xw1_ref[...] = _dot(x, w1_ref[...]) + b1_ref[...]


def _topk_kernel(et_ref, ef_ref, t_ref):
    sim = _dot(et_ref[...], ef_ref[...], trans_b=True)
    work = sim
    for _ in range(_KP1 - 1):
        m = jnp.max(work, axis=1, keepdims=True)
        work = jnp.where(work >= m, _NEG, work)
    t_ref[...] = jnp.max(work, axis=1, keepdims=True)


def _adj_kernel(et_ref, ef_ref, ti_ref, tj_ref, xw1_ref, w2_ref, b2_ref,
                adj_ref, t2_ref):
    sim = _dot(et_ref[...], ef_ref[...], trans_b=True)
    ti = ti_ref[...]
    tj = tj_ref[...]
    w = (sim >= ti).astype(jnp.float32) + (sim >= tj).astype(jnp.float32)
    adjt = 0.5 * jnp.maximum(sim, 0.0) * w
    adj_ref[...] = adjt
    h = jnp.maximum(_dot(adjt, xw1_ref[...]), 0.0)
    t2_ref[...] = _dot(h, w2_ref[...]) + b2_ref[...]


def _out_kernel(adj_ref, t2_ref, out_ref):
    out_ref[...] = _dot(adj_ref[...], t2_ref[...])


def _pick_tile(n, want):
    for r in (want, 400, 200, 100, 80, 40, 16, 8, 4, 2, 1):
        if r <= n and n % r == 0:
            return r
    return 1


def kernel(x, G_W1, G_b1, G_W2, G_b2, W1, b1, W2, b2):
    n, d = x.shape
    gh = G_W1.shape[1]
    h_dim = W1.shape[1]
    o_dim = W2.shape[1]

    gb1 = G_b1.reshape(1, gh)
    gb2 = G_b2.reshape(1, gh)
    b1r = b1.reshape(1, h_dim)
    b2r = b2.reshape(1, o_dim)

    full = lambda shape: pl.BlockSpec(shape, lambda i: (0, 0))

    r1 = _pick_tile(n, 1000)
    e, xw1 = pl.pallas_call(
        _embed_kernel,
        grid=(n // r1,),
        in_specs=[
            pl.BlockSpec((r1, d), lambda i: (i, 0)),
            full((d, gh)), full((1, gh)), full((gh, gh)), full((1, gh)),
            full((d, h_dim)), full((1, h_dim)),
        ],
        out_specs=[pl.BlockSpec((r1, gh), lambda i: (i, 0)),
                   pl.BlockSpec((r1, h_dim), lambda i: (i, 0))],
        out_shape=[jax.ShapeDtypeStruct((n, gh), jnp.float32),
                   jax.ShapeDtypeStruct((n, h_dim), jnp.float32)],
    )(x, G_W1, gb1, G_W2, gb2, W1, b1r)

    r2 = _pick_tile(n, 200)
    t = pl.pallas_call(
        _topk_kernel,
        grid=(n // r2,),
        in_specs=[pl.BlockSpec((r2, gh), lambda i: (i, 0)),
                  full((n, gh))],
        out_specs=pl.BlockSpec((r2, 1), lambda i: (i, 0)),
        out_shape=jax.ShapeDtypeStruct((n, 1), jnp.float32),
    )(e, e)
    t_row = t.reshape(1, n)

    r3 = _pick_tile(n, 200)
    adj, t2 = pl.pallas_call(
        _adj_kernel,
        grid=(n // r3,),
        in_specs=[
            pl.BlockSpec((r3, gh), lambda i: (i, 0)),
            full((n, gh)),
            pl.BlockSpec((r3, 1), lambda i: (i, 0)),
            full((1, n)),
            full((n, h_dim)),
            full((h_dim, o_dim)), full((1, o_dim)),
        ],
        out_specs=[pl.BlockSpec((r3, n), lambda i: (i, 0)),
                   pl.BlockSpec((r3, o_dim), lambda i: (i, 0))],
        out_shape=[jax.ShapeDtypeStruct((n, n), jnp.float32),
                   jax.ShapeDtypeStruct((n, o_dim), jnp.float32)],
    )(e, e, t, t_row, xw1, W2, b2r)

    r4 = _pick_tile(n, 400)
    out = pl.pallas_call(
        _out_kernel,
        grid=(n // r4,),
        in_specs=[pl.BlockSpec((r4, n), lambda i: (i, 0)),
                  full((n, o_dim))],
        out_specs=pl.BlockSpec((r4, o_dim), lambda i: (i, 0)),
        out_shape=jax.ShapeDtypeStruct((n, o_dim), jnp.float32),
    )(adj, t2)

    return (out, adj)
